# Initial kernel scaffold; baseline (speedup 1.0000x reference)
import jax
import jax.numpy as jnp
from jax.experimental import pallas as pl


def kernel(x, edge_index, edge_type, W1, Wr1, b1, W2, Wr2, b2, W3, Wr3, b3):
    raise NotImplementedError("write your pallas kernel here")


# speedup vs baseline: 13.2341x; 13.2341x over previous
import functools

import jax
import jax.numpy as jnp
from jax import lax
from jax.experimental import pallas as pl
from jax.experimental.pallas import tpu as pltpu
from jax.experimental.pallas import tpu_sc as plsc

N = 10000
E = 320000
D = 128
H = 128
R = 8

NC = 2
NS = 16
NW = NC * NS
CH = 128
E_PAD = 323584
EW = E_PAD // NW
NCH = EW // CH
NRP = 80128
CNT_SL = NRP // NS
ROWS_T = 632
ROWS_LAST = N - 15 * ROWS_T
NB = 25
BN = N // NB

_mesh = plsc.VectorSubcoreMesh(core_axis_name="c", subcore_axis_name="s")


def _wid():
    return lax.axis_index("s") * NC + lax.axis_index("c")


@functools.partial(
    pl.kernel,
    out_type=jax.ShapeDtypeStruct((NC * NRP,), jnp.float32),
    mesh=_mesh,
    scratch_types=[
        pltpu.VMEM_SHARED((NRP,), jnp.float32),
        pltpu.VMEM((CH,), jnp.int32),
        pltpu.VMEM((CH,), jnp.float32),
        pltpu.VMEM((CNT_SL,), jnp.float32),
    ],
)
def _sc_counts(dseg_hbm, cnt2_hbm, cnt_sp, idx_v, ones_v, zbuf):
    c = lax.axis_index("c")
    t = lax.axis_index("s")
    w = _wid()
    z16 = jnp.zeros((16,), jnp.float32)

    def zst(i, carry):
        zbuf[pl.ds(i * 16, 16)] = z16
        return carry

    lax.fori_loop(0, CNT_SL // 16, zst, 0)
    pltpu.sync_copy(zbuf, cnt_sp.at[pl.ds(t * CNT_SL, CNT_SL)])
    one16 = jnp.ones((16,), jnp.float32)
    for k in range(CH // 16):
        ones_v[pl.ds(k * 16, 16)] = one16
    plsc.subcore_barrier()

    def chunk(j, carry):
        base = w * EW + j * CH
        pltpu.sync_copy(dseg_hbm.at[pl.ds(base, CH)], idx_v)
        pltpu.sync_copy(ones_v, cnt_sp.at[idx_v], add=True)
        return carry

    lax.fori_loop(0, NCH, chunk, 0)
    plsc.subcore_barrier()
    pltpu.sync_copy(cnt_sp.at[pl.ds(t * CNT_SL, CNT_SL)], zbuf)
    pltpu.sync_copy(zbuf, cnt2_hbm.at[pl.ds(c * NRP + t * CNT_SL, CNT_SL)])


@functools.partial(
    pl.kernel,
    out_type=jax.ShapeDtypeStruct((E_PAD,), jnp.float32),
    mesh=_mesh,
    scratch_types=[
        pltpu.VMEM((CH,), jnp.int32),
        pltpu.VMEM((CH,), jnp.float32),
        pltpu.SemaphoreType.DMA,
    ],
)
def _sc_scales(dseg_hbm, inv_hbm, s_hbm, idx_v, sv, sem):
    w = _wid()

    def chunk(j, carry):
        base = w * EW + j * CH
        pltpu.sync_copy(dseg_hbm.at[pl.ds(base, CH)], idx_v)
        pltpu.async_copy(inv_hbm.at[idx_v], sv, sem).wait()
        pltpu.sync_copy(sv, s_hbm.at[pl.ds(base, CH)])
        return carry

    lax.fori_loop(0, NCH, chunk, 0)


@functools.partial(
    pl.kernel,
    out_type=jax.ShapeDtypeStruct((NC, N, H), jnp.float32),
    mesh=_mesh,
    scratch_types=[
        pltpu.VMEM_SHARED((N, H), jnp.float32),
        pltpu.VMEM((CH,), jnp.int32),
        pltpu.VMEM((CH,), jnp.int32),
        pltpu.VMEM((CH,), jnp.float32),
        pltpu.VMEM((CH, H), jnp.float32),
        pltpu.SemaphoreType.DMA,
    ],
)
def _sc_aggregate(h_hbm, g_hbm, dst_hbm, s_hbm, agg_hbm,
                  acc, gi, di, sv, rows, sem):
    c = lax.axis_index("c")
    t = lax.axis_index("s")
    w = _wid()
    off = t * ROWS_T
    z16 = jnp.zeros((16,), jnp.float32)

    def zst(i, carry):
        for k in range(H // 16):
            rows[i, pl.ds(k * 16, 16)] = z16
        return carry

    lax.fori_loop(0, CH, zst, 0)
    for j in range(4):
        pltpu.sync_copy(rows, acc.at[pl.ds(off + j * CH, CH)])

    @pl.when(t < NS - 1)
    def _():
        pltpu.sync_copy(rows.at[pl.ds(0, ROWS_T - 4 * CH)],
                        acc.at[pl.ds(off + 4 * CH, ROWS_T - 4 * CH)])

    @pl.when(t == NS - 1)
    def _():
        pltpu.sync_copy(rows.at[pl.ds(0, ROWS_LAST - 4 * CH)],
                        acc.at[pl.ds(off + 4 * CH, ROWS_LAST - 4 * CH)])

    plsc.subcore_barrier()

    def chunk(j, carry):
        base = w * EW + j * CH
        pltpu.sync_copy(g_hbm.at[pl.ds(base, CH)], gi)
        pltpu.sync_copy(dst_hbm.at[pl.ds(base, CH)], di)
        pltpu.sync_copy(s_hbm.at[pl.ds(base, CH)], sv)
        pltpu.async_copy(h_hbm.at[gi], rows, sem).wait()

        def egrp(gidx, cc):
            base_e = gidx * 16
            v = sv[pl.ds(base_e, 16)]
            for l in range(16):
                sb = v.at[jnp.full((16,), l, jnp.int32)].get(
                    mode='promise_in_bounds')
                e = base_e + l
                for k in range(H // 16):
                    rows[e, pl.ds(k * 16, 16)] = (
                        rows[e, pl.ds(k * 16, 16)] * sb)
            return cc

        lax.fori_loop(0, CH // 16, egrp, 0)
        pltpu.sync_copy(rows, acc.at[di], add=True)
        return carry

    lax.fori_loop(0, NCH, chunk, 0)
    plsc.subcore_barrier()
    for j in range(4):
        pltpu.sync_copy(acc.at[pl.ds(off + j * CH, CH)], rows)
        pltpu.sync_copy(rows, agg_hbm.at[c, pl.ds(off + j * CH, CH)])

    @pl.when(t < NS - 1)
    def _():
        tl = ROWS_T - 4 * CH
        pltpu.sync_copy(acc.at[pl.ds(off + 4 * CH, tl)],
                        rows.at[pl.ds(0, tl)])
        pltpu.sync_copy(rows.at[pl.ds(0, tl)],
                        agg_hbm.at[c, pl.ds(off + 4 * CH, tl)])

    @pl.when(t == NS - 1)
    def _():
        tl = ROWS_LAST - 4 * CH
        pltpu.sync_copy(acc.at[pl.ds(off + 4 * CH, tl)],
                        rows.at[pl.ds(0, tl)])
        pltpu.sync_copy(rows.at[pl.ds(0, tl)],
                        agg_hbm.at[c, pl.ds(off + 4 * CH, tl)])


def _inv_body(cnt2_ref, inv_ref):
    c = cnt2_ref[0] + cnt2_ref[1]
    rows = jax.lax.broadcasted_iota(jnp.int32, c.shape, 0)
    cols = jax.lax.broadcasted_iota(jnp.int32, c.shape, 1)
    flat = rows * 128 + cols
    inv = 1.0 / jnp.maximum(c, 1.0)
    inv_ref[...] = jnp.where(flat < N * R, inv, 0.0)


def _tc_inv(cnt2):
    out = pl.pallas_call(
        _inv_body,
        out_shape=jax.ShapeDtypeStruct((NRP // 128, 128), jnp.float32),
    )(cnt2.reshape(NC, NRP // 128, 128))
    return out.reshape(NRP)


def _mm_body(first, *refs):
    r = pl.program_id(1)
    if first:
        x_ref, w_ref, wr_ref, b_ref, h_ref, xr_ref = refs
        u = x_ref[...]
    else:
        agg_ref, xrp_ref, w_ref, wr_ref, b_ref, h_ref, xr_ref = refs
        u = jax.nn.relu(agg_ref[0] + agg_ref[1] + xrp_ref[...])
    h_ref[0] = jnp.dot(u, w_ref[0], preferred_element_type=jnp.float32)

    @pl.when(r == 0)
    def _():
        xr_ref[...] = (
            jnp.dot(u, wr_ref[...], preferred_element_type=jnp.float32)
            + b_ref[...]
        )


def _tc_matmul(u_args, W, Wr, b, first):
    if first:
        u_specs = [pl.BlockSpec((BN, D), lambda i, r: (i, 0))]
    else:
        u_specs = [
            pl.BlockSpec((NC, BN, H), lambda i, r: (0, i, 0)),
            pl.BlockSpec((BN, H), lambda i, r: (i, 0)),
        ]
    h, xr = pl.pallas_call(
        functools.partial(_mm_body, first),
        grid=(NB, R),
        in_specs=u_specs + [
            pl.BlockSpec((1, D, H), lambda i, r: (r, 0, 0)),
            pl.BlockSpec((D, H), lambda i, r: (0, 0)),
            pl.BlockSpec((1, H), lambda i, r: (0, 0)),
        ],
        out_specs=[
            pl.BlockSpec((1, BN, H), lambda i, r: (r, i, 0)),
            pl.BlockSpec((BN, H), lambda i, r: (i, 0)),
        ],
        out_shape=[
            jax.ShapeDtypeStruct((R, N, H), jnp.float32),
            jax.ShapeDtypeStruct((N, H), jnp.float32),
        ],
    )(*u_args, W, Wr, b.reshape(1, H))
    return h, xr


def _mean_body(agg_ref, xrp_ref, out_ref):
    i = pl.program_id(0)
    y = agg_ref[0] + agg_ref[1] + xrp_ref[...]
    part = jnp.sum(y, axis=0, keepdims=True)

    @pl.when(i == 0)
    def _():
        out_ref[...] = jnp.zeros_like(out_ref)

    out_ref[...] += part

    @pl.when(i == NB - 1)
    def _():
        out_ref[...] = out_ref[...] * (1.0 / N)


def _tc_mean(agg, xr):
    return pl.pallas_call(
        _mean_body,
        grid=(NB,),
        in_specs=[
            pl.BlockSpec((NC, BN, H), lambda i: (0, i, 0)),
            pl.BlockSpec((BN, H), lambda i: (i, 0)),
        ],
        out_specs=pl.BlockSpec((1, H), lambda i: (0, 0)),
        out_shape=jax.ShapeDtypeStruct((1, H), jnp.float32),
    )(agg, xr)


def kernel(x, edge_index, edge_type, W1, Wr1, b1, W2, Wr2, b2, W3, Wr3, b3):
    src = edge_index[0]
    dst = edge_index[1]
    et = edge_type
    g = et * N + src
    dseg = dst * R + et
    pad = E_PAD - E
    ar = jnp.arange(pad, dtype=jnp.int32)
    g_p = jnp.concatenate([g, ar % (N * R)])
    dseg_p = jnp.concatenate([dseg, N * R + (ar % 64)])
    dst_p = jnp.concatenate([dst, ar % N])

    cnt2 = _sc_counts(dseg_p)
    inv = _tc_inv(cnt2)
    s = _sc_scales(dseg_p, inv)

    h, xr = _tc_matmul((x,), W1, Wr1, b1, True)
    agg = _sc_aggregate(h.reshape(R * N, H), g_p, dst_p, s)

    h, xr = _tc_matmul((agg, xr), W2, Wr2, b2, False)
    agg = _sc_aggregate(h.reshape(R * N, H), g_p, dst_p, s)

    h, xr = _tc_matmul((agg, xr), W3, Wr3, b3, False)
    agg = _sc_aggregate(h.reshape(R * N, H), g_p, dst_p, s)

    return _tc_mean(agg, xr)
